# Initial kernel scaffold; baseline (speedup 1.0000x reference)
import jax
import jax.numpy as jnp
from jax.experimental import pallas as pl


def kernel(x, edge_index, W_w, W_b, a_w, a_b):
    raise NotImplementedError("write your pallas kernel here")


# speedup vs baseline: 1.9667x; 1.9667x over previous
import functools

import jax
import jax.numpy as jnp
from jax import lax
from jax.experimental import pallas as pl
from jax.experimental.pallas import tpu as pltpu
from jax.experimental.pallas import tpu_sc as plsc

NN = 10000
EE = 160000
DIN = 256
DP = 128
NH = 2

_BLK1 = 1000


def _k1_body(x_ref, w_ref, b_ref, ab_ref, wx0_ref, wx1_ref, s_ref, cs_ref):
    i = pl.program_id(0)
    xb = x_ref[...]
    wx = jnp.dot(xb, w_ref[...], preferred_element_type=jnp.float32) + b_ref[...]
    wx0_ref[...] = wx[:, :DP]
    wx1_ref[...] = wx[:, DP:]
    s_ref[...] = jnp.dot(wx, ab_ref[...], preferred_element_type=jnp.float32)
    colsum = jnp.sum(wx, axis=0, keepdims=True)

    @pl.when(i == 0)
    def _():
        cs_ref[...] = colsum

    @pl.when(i > 0)
    def _():
        cs_ref[...] += colsum


def _k1(x, wcat, bcat, ab):
    grid = NN // _BLK1
    return pl.pallas_call(
        _k1_body,
        grid=(grid,),
        in_specs=[
            pl.BlockSpec((_BLK1, DIN), lambda i: (i, 0)),
            pl.BlockSpec((DIN, NH * DP), lambda i: (0, 0)),
            pl.BlockSpec((1, NH * DP), lambda i: (0, 0)),
            pl.BlockSpec((DIN, 8), lambda i: (0, 0)),
        ],
        out_specs=[
            pl.BlockSpec((_BLK1, DP), lambda i: (i, 0)),
            pl.BlockSpec((_BLK1, DP), lambda i: (i, 0)),
            pl.BlockSpec((_BLK1, 8), lambda i: (i, 0)),
            pl.BlockSpec((1, NH * DP), lambda i: (0, 0)),
        ],
        out_shape=[
            jax.ShapeDtypeStruct((NN, DP), jnp.float32),
            jax.ShapeDtypeStruct((NN, DP), jnp.float32),
            jax.ShapeDtypeStruct((NN, 8), jnp.float32),
            jax.ShapeDtypeStruct((1, NH * DP), jnp.float32),
        ],
    )(x, wcat, bcat, ab)


_CH = EE // 16


def _k2a_body(sn_hbm, vi_hbm, vj_hbm, y0_hbm, y1_hbm, sn_v, vi_v, vj_v, y_v):
    c = lax.axis_index("c")
    s = lax.axis_index("s")
    base = s * _CH
    pltpu.sync_copy(sn_hbm, sn_v)
    pltpu.sync_copy(vi_hbm.at[pl.ds(base, _CH)], vi_v)
    pltpu.sync_copy(vj_hbm.at[pl.ds(base, _CH)], vj_v)
    zi = jnp.zeros((16,), jnp.int32)

    def body(k, _):
        sl = pl.ds(k * 16, 16)
        gi = plsc.load_gather(sn_v, [vi_v[sl] * 4 + c])
        gj = plsc.load_gather(sn_v, [vj_v[sl] * 4 + (NH + c)])
        t = gi + gj
        y_v[sl] = jnp.where(t >= 0.0, t, t * 0.2)
        return ()

    lax.fori_loop(0, _CH // 16, body, ())

    @pl.when(c == 0)
    def _():
        pltpu.sync_copy(y_v, y0_hbm.at[pl.ds(base, _CH)])

    @pl.when(c == 1)
    def _():
        pltpu.sync_copy(y_v, y1_hbm.at[pl.ds(base, _CH)])


def _k2a(sn, vi_s, vj_s):
    mesh = plsc.VectorSubcoreMesh(core_axis_name="c", subcore_axis_name="s")
    f = pl.kernel(
        _k2a_body,
        compiler_params=pltpu.CompilerParams(needs_layout_passes=False),
        out_type=[
            jax.ShapeDtypeStruct((EE,), jnp.float32),
            jax.ShapeDtypeStruct((EE,), jnp.float32),
        ],
        mesh=mesh,
        scratch_types=[
            pltpu.VMEM((NN * 2 * NH,), jnp.float32),
            pltpu.VMEM((_CH,), jnp.int32),
            pltpu.VMEM((_CH,), jnp.int32),
            pltpu.VMEM((_CH,), jnp.float32),
        ],
    )
    return f(sn, vi_s, vj_s)


_BB = 80
_NB = _CH // _BB
_ZPAD = 10240


def _k2b_body(wx_hbm, vi_hbm, vj_hbm, v0_hbm, v1_hbm, m_hbm, num_hbm,
              z0_hbm, z1_hbm,
              num_sh, z_sh, zbz_v, vi_v, vj_v, vja_v, vv_v, mv_v, wv_v,
              rows_v, sem):
    c = lax.axis_index("c")
    s = lax.axis_index("s")

    def zb_loop(k, _):
        rows_v[k // 8, pl.ds((k % 8) * 16, 16)] = jnp.zeros((16,), jnp.float32)
        return ()

    lax.fori_loop(0, _BB * 8, zb_loop, ())

    def zbz_loop(k, _):
        zbz_v[pl.ds(k * 16, 16)] = jnp.zeros((16,), jnp.float32)
        return ()

    lax.fori_loop(0, 40, zbz_loop, ())

    for mloop in range(5):
        pltpu.sync_copy(rows_v.at[pl.ds(0, 125)],
                        num_sh.at[pl.ds(s * 625 + mloop * 125, 125)])
    pltpu.sync_copy(zbz_v, z_sh.at[pl.ds(s * 640, 640)])
    plsc.subcore_barrier()

    def batch(b, _):
        base = s * _CH + b * _BB
        pltpu.sync_copy(vi_hbm.at[pl.ds(base, _BB)], vi_v)
        pltpu.sync_copy(vj_hbm.at[pl.ds(base, _BB)], vj_v)
        @pl.when(c == 0)
        def _():
            pltpu.sync_copy(v0_hbm.at[pl.ds(base, _BB)], vv_v)

        @pl.when(c == 1)
        def _():
            pltpu.sync_copy(v1_hbm.at[pl.ds(base, _BB)], vv_v)

        pltpu.sync_copy(m_hbm.at[pl.ds(base, _BB)], mv_v)

        def prep(k, _):
            sl = pl.ds(k * 16, 16)
            wv_v[sl] = mv_v[sl] * vv_v[sl]
            vja_v[sl] = vj_v[sl] + c * NN
            return ()

        lax.fori_loop(0, _BB // 16, prep, ())

        pltpu.sync_copy(wv_v, z_sh.at[vi_v], add=True)

        pltpu.async_copy(wx_hbm.at[vja_v], rows_v, sem).wait()

        def scale(r, _):
            wb = plsc.load_gather(wv_v, [jnp.zeros((16,), jnp.int32) + r])
            for p in range(DP // 16):
                sl = pl.ds(p * 16, 16)
                rows_v[r, sl] = rows_v[r, sl] * wb
            return ()

        lax.fori_loop(0, _BB, scale, ())

        pltpu.sync_copy(rows_v, num_sh.at[vi_v], add=True)
        return ()

    lax.fori_loop(0, _NB, batch, ())
    plsc.subcore_barrier()

    @pl.when((s == 0) & (c == 0))
    def _():
        pltpu.sync_copy(num_sh, num_hbm.at[0])
        pltpu.sync_copy(z_sh, z0_hbm)

    @pl.when((s == 0) & (c == 1))
    def _():
        pltpu.sync_copy(num_sh, num_hbm.at[1])
        pltpu.sync_copy(z_sh, z1_hbm)


def _k2b(wxcat, vi_s, vj_s, v0, v1, m):
    mesh = plsc.VectorSubcoreMesh(core_axis_name="c", subcore_axis_name="s")
    f = pl.kernel(
        _k2b_body,
        compiler_params=pltpu.CompilerParams(needs_layout_passes=False),
        out_type=[
            jax.ShapeDtypeStruct((NH, NN, DP), jnp.float32),
            jax.ShapeDtypeStruct((_ZPAD,), jnp.float32),
            jax.ShapeDtypeStruct((_ZPAD,), jnp.float32),
        ],
        mesh=mesh,
        scratch_types=[
            pltpu.VMEM_SHARED((NN, DP), jnp.float32),
            pltpu.VMEM_SHARED((_ZPAD,), jnp.float32),
            pltpu.VMEM((640,), jnp.float32),
            pltpu.VMEM((_BB,), jnp.int32),
            pltpu.VMEM((_BB,), jnp.int32),
            pltpu.VMEM((_BB,), jnp.int32),
            pltpu.VMEM((_BB,), jnp.float32),
            pltpu.VMEM((_BB,), jnp.float32),
            pltpu.VMEM((_BB,), jnp.float32),
            pltpu.VMEM((_BB, DP), jnp.float32),
            pltpu.SemaphoreType.DMA,
        ],
    )
    return f(wxcat, vi_s, vj_s, v0, v1, m)


_BLK3 = 1000


def _k3_body(num_ref, z_ref, cs_ref, o_ref):
    v = (cs_ref[0] + num_ref[0]) / (float(NN) + z_ref[0])
    o_ref[...] = jnp.where(v > 0.0, v, jnp.exp(v) - 1.0)


def _k3(num, z3, s2):
    return pl.pallas_call(
        _k3_body,
        grid=(NH, NN // _BLK3),
        in_specs=[
            pl.BlockSpec((1, _BLK3, DP), lambda h, i: (h, i, 0)),
            pl.BlockSpec((1, _BLK3, 1), lambda h, i: (h, i, 0)),
            pl.BlockSpec((1, 1, DP), lambda h, i: (h, 0, 0)),
        ],
        out_specs=pl.BlockSpec((_BLK3, DP), lambda h, i: (i, h)),
        out_shape=jax.ShapeDtypeStruct((NN, NH * DP), jnp.float32),
    )(num, z3, s2)


def kernel(x, edge_index, W_w, W_b, a_w, a_b):
    vi = edge_index[0]
    vj = edge_index[1]

    wcat = jnp.concatenate([W_w[0].T, W_w[1].T], axis=1)
    bcat = W_b.reshape(1, NH * DP)
    zcol = jnp.zeros((DP,), jnp.float32)
    ab = jnp.stack([
        jnp.concatenate([a_w[0, :DP], zcol]),
        jnp.concatenate([zcol, a_w[1, :DP]]),
        jnp.concatenate([a_w[0, DP:], zcol]),
        jnp.concatenate([zcol, a_w[1, DP:]]),
    ], axis=1)
    ab = jnp.concatenate([ab, jnp.zeros((NH * DP, 4), jnp.float32)], axis=1)

    wx0, wx1, s8, cs = _k1(x, wcat, bcat, ab)

    key = vi * NN + vj
    perm = jnp.argsort(key)
    ks = key[perm]
    vi_s = vi[perm]
    vj_s = vj[perm]

    sn = jnp.stack([
        s8[:, 0], s8[:, 1],
        s8[:, 2] + a_b[0], s8[:, 3] + a_b[1],
    ], axis=1)

    y0, y1 = _k2a(sn.reshape(-1), vi_s, vj_s)
    y2 = jnp.stack([y0, y1])

    starts = jnp.concatenate(
        [jnp.ones((1,), bool), ks[1:] != ks[:-1]])
    segid = jnp.cumsum(starts.astype(jnp.int32)) - 1
    vsum = jax.ops.segment_sum(y2.T, segid, num_segments=EE)
    v2 = vsum[segid].T
    m = starts.astype(jnp.float32)

    wxcat = jnp.concatenate([wx0, wx1], axis=0)
    w2 = jnp.exp(v2) - 1.0
    num, z0, z1 = _k2b(wxcat, vi_s, vj_s, w2[0], w2[1], m)

    z3 = jnp.stack([z0[:NN], z1[:NN]]).reshape(NH, NN, 1)
    s2 = cs.reshape(NH, 1, DP)
    return _k3(num, z3, s2)


# speedup vs baseline: 3.1149x; 1.5838x over previous
import functools

import jax
import jax.numpy as jnp
from jax import lax
from jax.experimental import pallas as pl
from jax.experimental.pallas import tpu as pltpu
from jax.experimental.pallas import tpu_sc as plsc

NN = 10000
EE = 160000
DIN = 256
DP = 128
NH = 2

_BLK1 = 1000


def _k1_body(x_ref, w_ref, b_ref, ab_ref, wx0_ref, wx1_ref, s_ref, cs_ref):
    i = pl.program_id(0)
    xb = x_ref[...]
    wx = jnp.dot(xb, w_ref[...], preferred_element_type=jnp.float32) + b_ref[...]
    wx0_ref[...] = wx[:, :DP]
    wx1_ref[...] = wx[:, DP:]
    s_ref[...] = jnp.dot(wx, ab_ref[...], preferred_element_type=jnp.float32)
    colsum = jnp.sum(wx, axis=0, keepdims=True)

    @pl.when(i == 0)
    def _():
        cs_ref[...] = colsum

    @pl.when(i > 0)
    def _():
        cs_ref[...] += colsum


def _k1(x, wcat, bcat, ab):
    grid = NN // _BLK1
    return pl.pallas_call(
        _k1_body,
        grid=(grid,),
        in_specs=[
            pl.BlockSpec((_BLK1, DIN), lambda i: (i, 0)),
            pl.BlockSpec((DIN, NH * DP), lambda i: (0, 0)),
            pl.BlockSpec((1, NH * DP), lambda i: (0, 0)),
            pl.BlockSpec((DIN, 8), lambda i: (0, 0)),
        ],
        out_specs=[
            pl.BlockSpec((_BLK1, DP), lambda i: (i, 0)),
            pl.BlockSpec((_BLK1, DP), lambda i: (i, 0)),
            pl.BlockSpec((_BLK1, 8), lambda i: (i, 0)),
            pl.BlockSpec((1, NH * DP), lambda i: (0, 0)),
        ],
        out_shape=[
            jax.ShapeDtypeStruct((NN, DP), jnp.float32),
            jax.ShapeDtypeStruct((NN, DP), jnp.float32),
            jax.ShapeDtypeStruct((NN, 8), jnp.float32),
            jax.ShapeDtypeStruct((1, NH * DP), jnp.float32),
        ],
    )(x, wcat, bcat, ab)


_CH = EE // 16


def _k2a_body(sn_hbm, vi_hbm, vj_hbm, y0_hbm, y1_hbm, sn_v, vi_v, vj_v, y_v):
    c = lax.axis_index("c")
    s = lax.axis_index("s")
    base = s * _CH
    pltpu.sync_copy(sn_hbm, sn_v)
    pltpu.sync_copy(vi_hbm.at[pl.ds(base, _CH)], vi_v)
    pltpu.sync_copy(vj_hbm.at[pl.ds(base, _CH)], vj_v)
    zi = jnp.zeros((16,), jnp.int32)

    def body(k, _):
        sl = pl.ds(k * 16, 16)
        gi = plsc.load_gather(sn_v, [vi_v[sl] * 4 + c])
        gj = plsc.load_gather(sn_v, [vj_v[sl] * 4 + (NH + c)])
        t = gi + gj
        y_v[sl] = jnp.where(t >= 0.0, t, t * 0.2)
        return ()

    lax.fori_loop(0, _CH // 16, body, ())

    @pl.when(c == 0)
    def _():
        pltpu.sync_copy(y_v, y0_hbm.at[pl.ds(base, _CH)])

    @pl.when(c == 1)
    def _():
        pltpu.sync_copy(y_v, y1_hbm.at[pl.ds(base, _CH)])


def _k2a(sn, vi_s, vj_s):
    mesh = plsc.VectorSubcoreMesh(core_axis_name="c", subcore_axis_name="s")
    f = pl.kernel(
        _k2a_body,
        compiler_params=pltpu.CompilerParams(needs_layout_passes=False),
        out_type=[
            jax.ShapeDtypeStruct((EE,), jnp.float32),
            jax.ShapeDtypeStruct((EE,), jnp.float32),
        ],
        mesh=mesh,
        scratch_types=[
            pltpu.VMEM((NN * 2 * NH,), jnp.float32),
            pltpu.VMEM((_CH,), jnp.int32),
            pltpu.VMEM((_CH,), jnp.int32),
            pltpu.VMEM((_CH,), jnp.float32),
        ],
    )
    return f(sn, vi_s, vj_s)


_BB = 80
_NB = _CH // _BB
_ZPAD = 10240


def _k2b_body(wx_hbm, vi_hbm, vj_hbm, v0_hbm, v1_hbm, m_hbm, num_hbm,
              z0_hbm, z1_hbm,
              num_sh, z_sh, zbz_v, vi_v, vj_v, vja_v, vv_v, mv_v, wv_v,
              rows_v, sem):
    c = lax.axis_index("c")
    s = lax.axis_index("s")

    def zb_loop(k, _):
        rows_v[k // 8, pl.ds((k % 8) * 16, 16)] = jnp.zeros((16,), jnp.float32)
        return ()

    lax.fori_loop(0, _BB * 8, zb_loop, ())

    def zbz_loop(k, _):
        zbz_v[pl.ds(k * 16, 16)] = jnp.zeros((16,), jnp.float32)
        return ()

    lax.fori_loop(0, 40, zbz_loop, ())

    for mloop in range(5):
        pltpu.sync_copy(rows_v.at[pl.ds(0, 125)],
                        num_sh.at[pl.ds(s * 625 + mloop * 125, 125)])
    pltpu.sync_copy(zbz_v, z_sh.at[pl.ds(s * 640, 640)])
    plsc.subcore_barrier()

    def batch(b, _):
        base = s * _CH + b * _BB
        pltpu.sync_copy(vi_hbm.at[pl.ds(base, _BB)], vi_v)
        pltpu.sync_copy(vj_hbm.at[pl.ds(base, _BB)], vj_v)
        @pl.when(c == 0)
        def _():
            pltpu.sync_copy(v0_hbm.at[pl.ds(base, _BB)], vv_v)

        @pl.when(c == 1)
        def _():
            pltpu.sync_copy(v1_hbm.at[pl.ds(base, _BB)], vv_v)

        pltpu.sync_copy(m_hbm.at[pl.ds(base, _BB)], mv_v)

        def prep(k, _):
            sl = pl.ds(k * 16, 16)
            wv_v[sl] = mv_v[sl] * vv_v[sl]
            vja_v[sl] = vj_v[sl] + c * NN
            return ()

        lax.fori_loop(0, _BB // 16, prep, ())

        pltpu.sync_copy(wv_v, z_sh.at[vi_v], add=True)

        pltpu.async_copy(wx_hbm.at[vja_v], rows_v, sem).wait()

        def scale(r, _):
            wb = plsc.load_gather(wv_v, [jnp.zeros((16,), jnp.int32) + r])
            for p in range(DP // 16):
                sl = pl.ds(p * 16, 16)
                rows_v[r, sl] = rows_v[r, sl] * wb
            return ()

        lax.fori_loop(0, _BB, scale, ())

        pltpu.sync_copy(rows_v, num_sh.at[vi_v], add=True)
        return ()

    lax.fori_loop(0, _NB, batch, ())
    plsc.subcore_barrier()

    @pl.when((s == 0) & (c == 0))
    def _():
        pltpu.sync_copy(num_sh, num_hbm.at[0])
        pltpu.sync_copy(z_sh, z0_hbm)

    @pl.when((s == 0) & (c == 1))
    def _():
        pltpu.sync_copy(num_sh, num_hbm.at[1])
        pltpu.sync_copy(z_sh, z1_hbm)


def _k2b(wxcat, vi_s, vj_s, v0, v1, m):
    mesh = plsc.VectorSubcoreMesh(core_axis_name="c", subcore_axis_name="s")
    f = pl.kernel(
        _k2b_body,
        compiler_params=pltpu.CompilerParams(needs_layout_passes=False),
        out_type=[
            jax.ShapeDtypeStruct((NH, NN, DP), jnp.float32),
            jax.ShapeDtypeStruct((_ZPAD,), jnp.float32),
            jax.ShapeDtypeStruct((_ZPAD,), jnp.float32),
        ],
        mesh=mesh,
        scratch_types=[
            pltpu.VMEM_SHARED((NN, DP), jnp.float32),
            pltpu.VMEM_SHARED((_ZPAD,), jnp.float32),
            pltpu.VMEM((640,), jnp.float32),
            pltpu.VMEM((_BB,), jnp.int32),
            pltpu.VMEM((_BB,), jnp.int32),
            pltpu.VMEM((_BB,), jnp.int32),
            pltpu.VMEM((_BB,), jnp.float32),
            pltpu.VMEM((_BB,), jnp.float32),
            pltpu.VMEM((_BB,), jnp.float32),
            pltpu.VMEM((_BB, DP), jnp.float32),
            pltpu.SemaphoreType.DMA,
        ],
    )
    return f(wxcat, vi_s, vj_s, v0, v1, m)


def _k2w_body(v_ref, m_ref, o_ref):
    o_ref[...] = m_ref[...] * (jnp.exp(v_ref[...]) - 1.0)


def _k2w(v2, m):
    vr = v2.reshape(20, NH * EE // 20)
    mr = jnp.broadcast_to(m, (NH, EE)).reshape(20, NH * EE // 20)
    out = pl.pallas_call(
        _k2w_body,
        out_shape=jax.ShapeDtypeStruct(vr.shape, jnp.float32),
    )(vr, mr)
    return out.reshape(NH, EE)


_BLK3 = 1000


def _k3_body(num_ref, z_ref, cs_ref, o_ref):
    v = (cs_ref[0] + num_ref[0]) / (float(NN) + z_ref[0])
    o_ref[...] = jnp.where(v > 0.0, v, jnp.exp(v) - 1.0)


def _k3(num, z3, s2):
    return pl.pallas_call(
        _k3_body,
        grid=(NH, NN // _BLK3),
        in_specs=[
            pl.BlockSpec((1, _BLK3, DP), lambda h, i: (h, i, 0)),
            pl.BlockSpec((1, _BLK3, 1), lambda h, i: (h, i, 0)),
            pl.BlockSpec((1, 1, DP), lambda h, i: (h, 0, 0)),
        ],
        out_specs=pl.BlockSpec((_BLK3, DP), lambda h, i: (i, h)),
        out_shape=jax.ShapeDtypeStruct((NN, NH * DP), jnp.float32),
    )(num, z3, s2)


def kernel(x, edge_index, W_w, W_b, a_w, a_b):
    vi = edge_index[0]
    vj = edge_index[1]

    wcat = jnp.concatenate([W_w[0].T, W_w[1].T], axis=1)
    bcat = W_b.reshape(1, NH * DP)
    zcol = jnp.zeros((DP,), jnp.float32)
    ab = jnp.stack([
        jnp.concatenate([a_w[0, :DP], zcol]),
        jnp.concatenate([zcol, a_w[1, :DP]]),
        jnp.concatenate([a_w[0, DP:], zcol]),
        jnp.concatenate([zcol, a_w[1, DP:]]),
    ], axis=1)
    ab = jnp.concatenate([ab, jnp.zeros((NH * DP, 4), jnp.float32)], axis=1)

    wx0, wx1, s8, cs = _k1(x, wcat, bcat, ab)

    key = vi * NN + vj
    perm = jnp.argsort(key)
    ks = key[perm]
    vi_s = vi[perm]
    vj_s = vj[perm]

    sn = jnp.stack([
        s8[:, 0], s8[:, 1],
        s8[:, 2] + a_b[0], s8[:, 3] + a_b[1],
    ], axis=1)

    y0, y1 = _k2a(sn.reshape(-1), vi_s, vj_s)
    y2 = jnp.stack([y0, y1])

    starts = jnp.concatenate(
        [jnp.ones((1,), bool), ks[1:] != ks[:-1]])
    lasts = jnp.concatenate([starts[1:], jnp.ones((1,), bool)])
    idxe = jnp.arange(EE, dtype=jnp.int32)
    c0 = jnp.cumsum(y2, axis=1)
    sidx = jax.lax.cummax(jnp.where(starts, idxe, 0))
    cprev = jnp.where(sidx > 0, c0[:, sidx - 1], 0.0)
    m = lasts.astype(jnp.float32)
    w2 = _k2w(c0 - cprev, m)

    wxcat = jnp.concatenate([wx0, wx1], axis=0)
    num, z0, z1 = _k2b(wxcat, vi_s, vj_s, w2[0], w2[1], m)

    z3 = jnp.stack([z0[:NN], z1[:NN]]).reshape(NH, NN, 1)
    s2 = cs.reshape(NH, 1, DP)
    return _k3(num, z3, s2)


# speedup vs baseline: 3.6509x; 1.1721x over previous
import functools

import jax
import jax.numpy as jnp
from jax import lax
from jax.experimental import pallas as pl
from jax.experimental.pallas import tpu as pltpu
from jax.experimental.pallas import tpu_sc as plsc

NN = 10000
EE = 160000
DIN = 256
DP = 128
NH = 2

_BLK1 = 1000


def _k1_body(x_ref, w_ref, b_ref, ab_ref, wx0_ref, wx1_ref, s_ref, cs_ref):
    i = pl.program_id(0)
    xb = x_ref[...]
    wx = jnp.dot(xb, w_ref[...], preferred_element_type=jnp.float32) + b_ref[...]
    wx0_ref[...] = wx[:, :DP]
    wx1_ref[...] = wx[:, DP:]
    s_ref[...] = jnp.dot(wx, ab_ref[...], preferred_element_type=jnp.float32)
    colsum = jnp.sum(wx, axis=0, keepdims=True)

    @pl.when(i == 0)
    def _():
        cs_ref[...] = colsum

    @pl.when(i > 0)
    def _():
        cs_ref[...] += colsum


def _k1(x, wcat, bcat, ab):
    grid = NN // _BLK1
    return pl.pallas_call(
        _k1_body,
        grid=(grid,),
        in_specs=[
            pl.BlockSpec((_BLK1, DIN), lambda i: (i, 0)),
            pl.BlockSpec((DIN, NH * DP), lambda i: (0, 0)),
            pl.BlockSpec((1, NH * DP), lambda i: (0, 0)),
            pl.BlockSpec((DIN, 8), lambda i: (0, 0)),
        ],
        out_specs=[
            pl.BlockSpec((_BLK1, DP), lambda i: (i, 0)),
            pl.BlockSpec((_BLK1, DP), lambda i: (i, 0)),
            pl.BlockSpec((_BLK1, 8), lambda i: (i, 0)),
            pl.BlockSpec((1, NH * DP), lambda i: (0, 0)),
        ],
        out_shape=[
            jax.ShapeDtypeStruct((NN, DP), jnp.float32),
            jax.ShapeDtypeStruct((NN, DP), jnp.float32),
            jax.ShapeDtypeStruct((NN, 8), jnp.float32),
            jax.ShapeDtypeStruct((1, NH * DP), jnp.float32),
        ],
    )(x, wcat, bcat, ab)


_CH = EE // 16


def _k2a_body(sn_hbm, vi_hbm, vj_hbm, y0_hbm, y1_hbm, sn_v, vi_v, vj_v, y_v):
    c = lax.axis_index("c")
    s = lax.axis_index("s")
    base = s * _CH
    pltpu.sync_copy(sn_hbm, sn_v)
    pltpu.sync_copy(vi_hbm.at[pl.ds(base, _CH)], vi_v)
    pltpu.sync_copy(vj_hbm.at[pl.ds(base, _CH)], vj_v)
    zi = jnp.zeros((16,), jnp.int32)

    def body(k, _):
        sl = pl.ds(k * 16, 16)
        gi = plsc.load_gather(sn_v, [vi_v[sl] * 4 + c])
        gj = plsc.load_gather(sn_v, [vj_v[sl] * 4 + (NH + c)])
        t = gi + gj
        y_v[sl] = jnp.where(t >= 0.0, t, t * 0.2)
        return ()

    lax.fori_loop(0, _CH // 16, body, ())

    @pl.when(c == 0)
    def _():
        pltpu.sync_copy(y_v, y0_hbm.at[pl.ds(base, _CH)])

    @pl.when(c == 1)
    def _():
        pltpu.sync_copy(y_v, y1_hbm.at[pl.ds(base, _CH)])


def _k2a(sn, vi_s, vj_s):
    mesh = plsc.VectorSubcoreMesh(core_axis_name="c", subcore_axis_name="s")
    f = pl.kernel(
        _k2a_body,
        compiler_params=pltpu.CompilerParams(needs_layout_passes=False),
        out_type=[
            jax.ShapeDtypeStruct((EE,), jnp.float32),
            jax.ShapeDtypeStruct((EE,), jnp.float32),
        ],
        mesh=mesh,
        scratch_types=[
            pltpu.VMEM((NN * 2 * NH,), jnp.float32),
            pltpu.VMEM((_CH,), jnp.int32),
            pltpu.VMEM((_CH,), jnp.int32),
            pltpu.VMEM((_CH,), jnp.float32),
        ],
    )
    return f(sn, vi_s, vj_s)


_BB = 80
_MB = 8
_MC = 16
_RPT = _MB * _MC
_EPW = _RPT * _BB
_ZPAD = 10240


def _k2b_body(wx_hbm, vi2_hbm, vj2_hbm, w0_hbm, w1_hbm, num_hbm, z0_hbm,
              z1_hbm, num_sh, z_sh, zbz_v, vi2_v, vj2_v, vja_v, wv_v,
              rows_a, rows_b, sem_a, sem_b):
    c = lax.axis_index("c")
    s = lax.axis_index("s")
    rows = (rows_a, rows_b)
    sems = (sem_a, sem_b)

    def za_loop(k, _):
        rows_a[k // 8, pl.ds((k % 8) * 16, 16)] = jnp.zeros((16,), jnp.float32)
        return ()

    lax.fori_loop(0, _BB * 8, za_loop, ())

    def zbz_loop(k, _):
        zbz_v[pl.ds(k * 16, 16)] = jnp.zeros((16,), jnp.float32)
        return ()

    lax.fori_loop(0, 40, zbz_loop, ())

    for q in range(7):
        pltpu.sync_copy(rows_a, num_sh.at[pl.ds(s * 625 + q * 80, 80)])
    pltpu.sync_copy(rows_a.at[pl.ds(0, 65)],
                    num_sh.at[pl.ds(s * 625 + 560, 65)])
    pltpu.sync_copy(zbz_v, z_sh.at[pl.ds(s * 640, 640)])
    plsc.subcore_barrier()

    def scale(rref, wofs):
        def body(r, _):
            wb = plsc.load_gather(wv_v, [jnp.zeros((16,), jnp.int32) + wofs + r])
            for p in range(DP // 16):
                sl = pl.ds(p * 16, 16)
                rref[r, sl] = rref[r, sl] * wb
            return ()

        lax.fori_loop(0, _BB, body, ())

    def mc_loop(mc, _):
        rowbase = s * _RPT + mc * _MB
        base = rowbase * _BB
        pltpu.sync_copy(vi2_hbm.at[pl.ds(rowbase, _MB)], vi2_v)
        pltpu.sync_copy(vj2_hbm.at[pl.ds(rowbase, _MB)], vj2_v)

        @pl.when(c == 0)
        def _():
            pltpu.sync_copy(w0_hbm.at[pl.ds(base, _MB * _BB)], wv_v)

        @pl.when(c == 1)
        def _():
            pltpu.sync_copy(w1_hbm.at[pl.ds(base, _MB * _BB)], wv_v)

        def prep(k, _):
            sl = pl.ds((k % 5) * 16, 16)
            vja_v[k // 5, sl] = vj2_v[k // 5, sl] + c * NN
            return ()

        lax.fori_loop(0, _MB * 5, prep, ())

        pltpu.async_copy(wx_hbm.at[vja_v.at[0]], rows_a, sem_a)
        for b in range(_MB):
            pb = b % 2
            if b + 1 < _MB:
                pltpu.async_copy(wx_hbm.at[vja_v.at[b + 1]],
                                 rows[(b + 1) % 2], sems[(b + 1) % 2])
            pltpu.make_async_copy(wx_hbm.at[vja_v.at[b]],
                                  rows[pb], sems[pb]).wait()
            scale(rows[pb], b * _BB)
            pltpu.sync_copy(rows[pb], num_sh.at[vi2_v.at[b]], add=True)
            pltpu.sync_copy(wv_v.at[pl.ds(b * _BB, _BB)],
                            z_sh.at[vi2_v.at[b]], add=True)
        return ()

    lax.fori_loop(0, _MC, mc_loop, ())
    plsc.subcore_barrier()

    @pl.when((s == 0) & (c == 0))
    def _():
        pltpu.sync_copy(num_sh, num_hbm.at[0])
        pltpu.sync_copy(z_sh, z0_hbm)

    @pl.when((s == 0) & (c == 1))
    def _():
        pltpu.sync_copy(num_sh, num_hbm.at[1])
        pltpu.sync_copy(z_sh, z1_hbm)


def _k2b(wxcat, vi2, vj2, w0, w1):
    mesh = plsc.VectorSubcoreMesh(core_axis_name="c", subcore_axis_name="s")
    f = pl.kernel(
        _k2b_body,
        compiler_params=pltpu.CompilerParams(needs_layout_passes=False),
        out_type=[
            jax.ShapeDtypeStruct((NH, NN, DP), jnp.float32),
            jax.ShapeDtypeStruct((_ZPAD,), jnp.float32),
            jax.ShapeDtypeStruct((_ZPAD,), jnp.float32),
        ],
        mesh=mesh,
        scratch_types=[
            pltpu.VMEM_SHARED((NN, DP), jnp.float32),
            pltpu.VMEM_SHARED((_ZPAD,), jnp.float32),
            pltpu.VMEM((640,), jnp.float32),
            pltpu.VMEM((_MB, _BB), jnp.int32),
            pltpu.VMEM((_MB, _BB), jnp.int32),
            pltpu.VMEM((_MB, _BB), jnp.int32),
            pltpu.VMEM((_MB * _BB,), jnp.float32),
            pltpu.VMEM((_BB, DP), jnp.float32),
            pltpu.VMEM((_BB, DP), jnp.float32),
            pltpu.SemaphoreType.DMA,
            pltpu.SemaphoreType.DMA,
        ],
    )
    return f(wxcat, vi2, vj2, w0, w1)


def _k2w_body(v_ref, m_ref, o_ref):
    o_ref[...] = m_ref[...] * (jnp.exp(v_ref[...]) - 1.0)


def _k2w(v2, m):
    vr = v2.reshape(20, NH * EE // 20)
    mr = jnp.broadcast_to(m, (NH, EE)).reshape(20, NH * EE // 20)
    out = pl.pallas_call(
        _k2w_body,
        out_shape=jax.ShapeDtypeStruct(vr.shape, jnp.float32),
    )(vr, mr)
    return out.reshape(NH, EE)


_BLK3 = 1000


def _k3_body(num_ref, z_ref, cs_ref, o_ref):
    v = (cs_ref[0] + num_ref[0]) / (float(NN) + z_ref[0])
    o_ref[...] = jnp.where(v > 0.0, v, jnp.exp(v) - 1.0)


def _k3(num, z3, s2):
    return pl.pallas_call(
        _k3_body,
        grid=(NH, NN // _BLK3),
        in_specs=[
            pl.BlockSpec((1, _BLK3, DP), lambda h, i: (h, i, 0)),
            pl.BlockSpec((1, _BLK3, 1), lambda h, i: (h, i, 0)),
            pl.BlockSpec((1, 1, DP), lambda h, i: (h, 0, 0)),
        ],
        out_specs=pl.BlockSpec((_BLK3, DP), lambda h, i: (i, h)),
        out_shape=jax.ShapeDtypeStruct((NN, NH * DP), jnp.float32),
    )(num, z3, s2)


def kernel(x, edge_index, W_w, W_b, a_w, a_b):
    vi = edge_index[0]
    vj = edge_index[1]

    wcat = jnp.concatenate([W_w[0].T, W_w[1].T], axis=1)
    bcat = W_b.reshape(1, NH * DP)
    zcol = jnp.zeros((DP,), jnp.float32)
    ab = jnp.stack([
        jnp.concatenate([a_w[0, :DP], zcol]),
        jnp.concatenate([zcol, a_w[1, :DP]]),
        jnp.concatenate([a_w[0, DP:], zcol]),
        jnp.concatenate([zcol, a_w[1, DP:]]),
    ], axis=1)
    ab = jnp.concatenate([ab, jnp.zeros((NH * DP, 4), jnp.float32)], axis=1)

    wx0, wx1, s8, cs = _k1(x, wcat, bcat, ab)

    key = vi * NN + vj
    perm = jnp.argsort(key)
    ks = key[perm]
    vi_s = vi[perm]
    vj_s = vj[perm]

    sn = jnp.stack([
        s8[:, 0], s8[:, 1],
        s8[:, 2] + a_b[0], s8[:, 3] + a_b[1],
    ], axis=1)

    y0, y1 = _k2a(sn.reshape(-1), vi_s, vj_s)
    y2 = jnp.stack([y0, y1])

    starts = jnp.concatenate(
        [jnp.ones((1,), bool), ks[1:] != ks[:-1]])
    lasts = jnp.concatenate([starts[1:], jnp.ones((1,), bool)])
    idxe = jnp.arange(EE, dtype=jnp.int32)
    c0 = jnp.cumsum(y2, axis=1)
    sidx = jax.lax.cummax(jnp.where(starts, idxe, 0))
    cprev = jnp.where(sidx > 0, c0[:, sidx - 1], 0.0)
    m = lasts.astype(jnp.float32)
    w2 = _k2w(c0 - cprev, m)

    wxcat = jnp.concatenate([wx0, wx1], axis=0)
    padi = jnp.zeros((16, 3, _BB), jnp.int32)
    padw = jnp.zeros((NH, 16, 3, _BB), jnp.float32)
    vi2 = jnp.concatenate(
        [vi_s.reshape(16, 125, _BB), padi], axis=1).reshape(16 * 128, _BB)
    vj2 = jnp.concatenate(
        [vj_s.reshape(16, 125, _BB), padi], axis=1).reshape(16 * 128, _BB)
    w2p = jnp.concatenate(
        [w2.reshape(NH, 16, 125, _BB), padw], axis=2).reshape(NH, 16 * 128 * _BB)
    num, z0, z1 = _k2b(wxcat, vi2, vj2, w2p[0], w2p[1])

    z3 = jnp.stack([z0[:NN], z1[:NN]]).reshape(NH, NN, 1)
    s2 = cs.reshape(NH, 1, DP)
    return _k3(num, z3, s2)


# speedup vs baseline: 3.7725x; 1.0333x over previous
import functools

import jax
import jax.numpy as jnp
from jax import lax
from jax.experimental import pallas as pl
from jax.experimental.pallas import tpu as pltpu
from jax.experimental.pallas import tpu_sc as plsc

NN = 10000
EE = 160000
DIN = 256
DP = 128
NH = 2

_BLK1 = 1000


def _k1_body(x_ref, w_ref, b_ref, ab_ref, wx0_ref, wx1_ref, s_ref, cs_ref):
    i = pl.program_id(0)
    xb = x_ref[...]
    wx = jnp.dot(xb, w_ref[...], preferred_element_type=jnp.float32) + b_ref[...]
    wx0_ref[...] = wx[:, :DP]
    wx1_ref[...] = wx[:, DP:]
    s_ref[...] = jnp.dot(wx, ab_ref[...], preferred_element_type=jnp.float32)
    colsum = jnp.sum(wx, axis=0, keepdims=True)

    @pl.when(i == 0)
    def _():
        cs_ref[...] = colsum

    @pl.when(i > 0)
    def _():
        cs_ref[...] += colsum


def _k1(x, wcat, bcat, ab):
    grid = NN // _BLK1
    return pl.pallas_call(
        _k1_body,
        grid=(grid,),
        in_specs=[
            pl.BlockSpec((_BLK1, DIN), lambda i: (i, 0)),
            pl.BlockSpec((DIN, NH * DP), lambda i: (0, 0)),
            pl.BlockSpec((1, NH * DP), lambda i: (0, 0)),
            pl.BlockSpec((DIN, 8), lambda i: (0, 0)),
        ],
        out_specs=[
            pl.BlockSpec((_BLK1, DP), lambda i: (i, 0)),
            pl.BlockSpec((_BLK1, DP), lambda i: (i, 0)),
            pl.BlockSpec((_BLK1, 8), lambda i: (i, 0)),
            pl.BlockSpec((1, NH * DP), lambda i: (0, 0)),
        ],
        out_shape=[
            jax.ShapeDtypeStruct((NN, DP), jnp.float32),
            jax.ShapeDtypeStruct((NN, DP), jnp.float32),
            jax.ShapeDtypeStruct((NN, 8), jnp.float32),
            jax.ShapeDtypeStruct((1, NH * DP), jnp.float32),
        ],
    )(x, wcat, bcat, ab)


_CH = EE // 16


def _k2a_body(sn_hbm, vi_hbm, vj_hbm, y0_hbm, y1_hbm, sn_v, vi_v, vj_v, y_v):
    c = lax.axis_index("c")
    s = lax.axis_index("s")
    base = s * _CH
    pltpu.sync_copy(sn_hbm, sn_v)
    pltpu.sync_copy(vi_hbm.at[pl.ds(base, _CH)], vi_v)
    pltpu.sync_copy(vj_hbm.at[pl.ds(base, _CH)], vj_v)
    zi = jnp.zeros((16,), jnp.int32)

    def body(k, _):
        sl = pl.ds(k * 16, 16)
        gi = plsc.load_gather(sn_v, [vi_v[sl] * 4 + c])
        gj = plsc.load_gather(sn_v, [vj_v[sl] * 4 + (NH + c)])
        t = gi + gj
        y_v[sl] = jnp.where(t >= 0.0, t, t * 0.2)
        return ()

    lax.fori_loop(0, _CH // 16, body, ())

    @pl.when(c == 0)
    def _():
        pltpu.sync_copy(y_v, y0_hbm.at[pl.ds(base, _CH)])

    @pl.when(c == 1)
    def _():
        pltpu.sync_copy(y_v, y1_hbm.at[pl.ds(base, _CH)])


def _k2a(sn, vi_s, vj_s):
    mesh = plsc.VectorSubcoreMesh(core_axis_name="c", subcore_axis_name="s")
    f = pl.kernel(
        _k2a_body,
        compiler_params=pltpu.CompilerParams(needs_layout_passes=False),
        out_type=[
            jax.ShapeDtypeStruct((EE,), jnp.float32),
            jax.ShapeDtypeStruct((EE,), jnp.float32),
        ],
        mesh=mesh,
        scratch_types=[
            pltpu.VMEM((NN * 2 * NH,), jnp.float32),
            pltpu.VMEM((_CH,), jnp.int32),
            pltpu.VMEM((_CH,), jnp.int32),
            pltpu.VMEM((_CH,), jnp.float32),
        ],
    )
    return f(sn, vi_s, vj_s)


_BB = 80
_MB = 8
_MC = 16
_RPT = _MB * _MC
_EPW = _RPT * _BB
_ZPAD = 10240


def _k2b_body(wx_hbm, vi2_hbm, vj2_hbm, w0_hbm, w1_hbm, num_hbm, z0_hbm,
              z1_hbm, num_sh, z_sh, zbz_v, vi2_v, vj2_v, vja_v, wv_v,
              rows_a, rows_b, sem_a, sem_b):
    c = lax.axis_index("c")
    s = lax.axis_index("s")
    rows = (rows_a, rows_b)
    sems = (sem_a, sem_b)

    def za_loop(k, _):
        rows_a[k // 8, pl.ds((k % 8) * 16, 16)] = jnp.zeros((16,), jnp.float32)
        return ()

    lax.fori_loop(0, _BB * 8, za_loop, ())

    def zbz_loop(k, _):
        zbz_v[pl.ds(k * 16, 16)] = jnp.zeros((16,), jnp.float32)
        return ()

    lax.fori_loop(0, 40, zbz_loop, ())

    for q in range(7):
        pltpu.sync_copy(rows_a, num_sh.at[pl.ds(s * 625 + q * 80, 80)])
    pltpu.sync_copy(rows_a.at[pl.ds(0, 65)],
                    num_sh.at[pl.ds(s * 625 + 560, 65)])
    pltpu.sync_copy(zbz_v, z_sh.at[pl.ds(s * 640, 640)])
    plsc.subcore_barrier()

    def scale(rref, wofs):
        def body(rr, _):
            r = rr * 2
            zi = jnp.zeros((16,), jnp.int32)
            wb0 = plsc.load_gather(wv_v, [zi + wofs + r])
            wb1 = plsc.load_gather(wv_v, [zi + wofs + r + 1])
            for p in range(DP // 16):
                sl = pl.ds(p * 16, 16)
                rref[r, sl] = rref[r, sl] * wb0
                rref[r + 1, sl] = rref[r + 1, sl] * wb1
            return ()

        lax.fori_loop(0, _BB // 2, body, ())

    def mc_loop(mc, _):
        rowbase = s * _RPT + mc * _MB
        base = rowbase * _BB
        pltpu.sync_copy(vi2_hbm.at[pl.ds(rowbase, _MB)], vi2_v)
        pltpu.sync_copy(vj2_hbm.at[pl.ds(rowbase, _MB)], vj2_v)

        @pl.when(c == 0)
        def _():
            pltpu.sync_copy(w0_hbm.at[pl.ds(base, _MB * _BB)], wv_v)

        @pl.when(c == 1)
        def _():
            pltpu.sync_copy(w1_hbm.at[pl.ds(base, _MB * _BB)], wv_v)

        def prep(k, _):
            sl = pl.ds((k % 5) * 16, 16)
            vja_v[k // 5, sl] = vj2_v[k // 5, sl] + c * NN
            return ()

        lax.fori_loop(0, _MB * 5, prep, ())

        pltpu.async_copy(wx_hbm.at[vja_v.at[0]], rows_a, sem_a)
        for b in range(_MB):
            pb = b % 2
            if b + 1 < _MB:
                pltpu.async_copy(wx_hbm.at[vja_v.at[b + 1]],
                                 rows[(b + 1) % 2], sems[(b + 1) % 2])
            pltpu.make_async_copy(wx_hbm.at[vja_v.at[b]],
                                  rows[pb], sems[pb]).wait()
            scale(rows[pb], b * _BB)
            pltpu.sync_copy(rows[pb], num_sh.at[vi2_v.at[b]], add=True)
            pltpu.sync_copy(wv_v.at[pl.ds(b * _BB, _BB)],
                            z_sh.at[vi2_v.at[b]], add=True)
        return ()

    lax.fori_loop(0, _MC, mc_loop, ())
    plsc.subcore_barrier()

    @pl.when((s == 0) & (c == 0))
    def _():
        pltpu.sync_copy(num_sh, num_hbm.at[0])
        pltpu.sync_copy(z_sh, z0_hbm)

    @pl.when((s == 0) & (c == 1))
    def _():
        pltpu.sync_copy(num_sh, num_hbm.at[1])
        pltpu.sync_copy(z_sh, z1_hbm)


def _k2b(wxcat, vi2, vj2, w0, w1):
    mesh = plsc.VectorSubcoreMesh(core_axis_name="c", subcore_axis_name="s")
    f = pl.kernel(
        _k2b_body,
        compiler_params=pltpu.CompilerParams(needs_layout_passes=False),
        out_type=[
            jax.ShapeDtypeStruct((NH, NN, DP), jnp.float32),
            jax.ShapeDtypeStruct((_ZPAD,), jnp.float32),
            jax.ShapeDtypeStruct((_ZPAD,), jnp.float32),
        ],
        mesh=mesh,
        scratch_types=[
            pltpu.VMEM_SHARED((NN, DP), jnp.float32),
            pltpu.VMEM_SHARED((_ZPAD,), jnp.float32),
            pltpu.VMEM((640,), jnp.float32),
            pltpu.VMEM((_MB, _BB), jnp.int32),
            pltpu.VMEM((_MB, _BB), jnp.int32),
            pltpu.VMEM((_MB, _BB), jnp.int32),
            pltpu.VMEM((_MB * _BB,), jnp.float32),
            pltpu.VMEM((_BB, DP), jnp.float32),
            pltpu.VMEM((_BB, DP), jnp.float32),
            pltpu.SemaphoreType.DMA,
            pltpu.SemaphoreType.DMA,
        ],
    )
    return f(wxcat, vi2, vj2, w0, w1)


def _k2w_body(v_ref, m_ref, o_ref):
    o_ref[...] = m_ref[...] * (jnp.exp(v_ref[...]) - 1.0)


def _k2w(v2, m):
    vr = v2.reshape(20, NH * EE // 20)
    mr = jnp.broadcast_to(m, (NH, EE)).reshape(20, NH * EE // 20)
    out = pl.pallas_call(
        _k2w_body,
        out_shape=jax.ShapeDtypeStruct(vr.shape, jnp.float32),
    )(vr, mr)
    return out.reshape(NH, EE)


_BLK3 = 1000


def _k3_body(num_ref, z_ref, cs_ref, o_ref):
    v = (cs_ref[0] + num_ref[0]) / (float(NN) + z_ref[0])
    o_ref[...] = jnp.where(v > 0.0, v, jnp.exp(v) - 1.0)


def _k3(num, z3, s2):
    return pl.pallas_call(
        _k3_body,
        grid=(NH, NN // _BLK3),
        in_specs=[
            pl.BlockSpec((1, _BLK3, DP), lambda h, i: (h, i, 0)),
            pl.BlockSpec((1, _BLK3, 1), lambda h, i: (h, i, 0)),
            pl.BlockSpec((1, 1, DP), lambda h, i: (h, 0, 0)),
        ],
        out_specs=pl.BlockSpec((_BLK3, DP), lambda h, i: (i, h)),
        out_shape=jax.ShapeDtypeStruct((NN, NH * DP), jnp.float32),
    )(num, z3, s2)


def kernel(x, edge_index, W_w, W_b, a_w, a_b):
    vi = edge_index[0]
    vj = edge_index[1]

    wcat = jnp.concatenate([W_w[0].T, W_w[1].T], axis=1)
    bcat = W_b.reshape(1, NH * DP)
    zcol = jnp.zeros((DP,), jnp.float32)
    ab = jnp.stack([
        jnp.concatenate([a_w[0, :DP], zcol]),
        jnp.concatenate([zcol, a_w[1, :DP]]),
        jnp.concatenate([a_w[0, DP:], zcol]),
        jnp.concatenate([zcol, a_w[1, DP:]]),
    ], axis=1)
    ab = jnp.concatenate([ab, jnp.zeros((NH * DP, 4), jnp.float32)], axis=1)

    wx0, wx1, s8, cs = _k1(x, wcat, bcat, ab)

    key = vi * NN + vj
    perm = jnp.argsort(key)
    ks = key[perm]
    vi_s = vi[perm]
    vj_s = vj[perm]

    sn = jnp.stack([
        s8[:, 0], s8[:, 1],
        s8[:, 2] + a_b[0], s8[:, 3] + a_b[1],
    ], axis=1)

    y0, y1 = _k2a(sn.reshape(-1), vi_s, vj_s)
    y2 = jnp.stack([y0, y1])

    starts = jnp.concatenate(
        [jnp.ones((1,), bool), ks[1:] != ks[:-1]])
    lasts = jnp.concatenate([starts[1:], jnp.ones((1,), bool)])
    idxe = jnp.arange(EE, dtype=jnp.int32)
    c0 = jnp.cumsum(y2, axis=1)
    sidx = jax.lax.cummax(jnp.where(starts, idxe, 0))
    cprev = jnp.where(sidx > 0, c0[:, sidx - 1], 0.0)
    m = lasts.astype(jnp.float32)
    w2 = _k2w(c0 - cprev, m)

    wxcat = jnp.concatenate([wx0, wx1], axis=0)
    padi = jnp.zeros((16, 3, _BB), jnp.int32)
    padw = jnp.zeros((NH, 16, 3, _BB), jnp.float32)
    vi2 = jnp.concatenate(
        [vi_s.reshape(16, 125, _BB), padi], axis=1).reshape(16 * 128, _BB)
    vj2 = jnp.concatenate(
        [vj_s.reshape(16, 125, _BB), padi], axis=1).reshape(16 * 128, _BB)
    w2p = jnp.concatenate(
        [w2.reshape(NH, 16, 125, _BB), padw], axis=2).reshape(NH, 16 * 128 * _BB)
    num, z0, z1 = _k2b(wxcat, vi2, vj2, w2p[0], w2p[1])

    z3 = jnp.stack([z0[:NN], z1[:NN]]).reshape(NH, NN, 1)
    s2 = cs.reshape(NH, 1, DP)
    return _k3(num, z3, s2)


# speedup vs baseline: 3.9524x; 1.0477x over previous
import functools

import jax
import jax.numpy as jnp
from jax import lax
from jax.experimental import pallas as pl
from jax.experimental.pallas import tpu as pltpu
from jax.experimental.pallas import tpu_sc as plsc

NN = 10000
EE = 160000
DIN = 256
DP = 128
NH = 2

_BLK1 = 1000


def _k1_body(x_ref, w_ref, b_ref, ab_ref, wx0_ref, wx1_ref, s_ref, cs_ref):
    i = pl.program_id(0)
    xb = x_ref[...]
    wx = jnp.dot(xb, w_ref[...], preferred_element_type=jnp.float32) + b_ref[...]
    wx0_ref[...] = wx[:, :DP]
    wx1_ref[...] = wx[:, DP:]
    s_ref[...] = jnp.dot(wx, ab_ref[...], preferred_element_type=jnp.float32)
    colsum = jnp.sum(wx, axis=0, keepdims=True)

    @pl.when(i == 0)
    def _():
        cs_ref[...] = colsum

    @pl.when(i > 0)
    def _():
        cs_ref[...] += colsum


def _k1(x, wcat, bcat, ab):
    grid = NN // _BLK1
    return pl.pallas_call(
        _k1_body,
        grid=(grid,),
        in_specs=[
            pl.BlockSpec((_BLK1, DIN), lambda i: (i, 0)),
            pl.BlockSpec((DIN, NH * DP), lambda i: (0, 0)),
            pl.BlockSpec((1, NH * DP), lambda i: (0, 0)),
            pl.BlockSpec((DIN, 8), lambda i: (0, 0)),
        ],
        out_specs=[
            pl.BlockSpec((_BLK1, DP), lambda i: (i, 0)),
            pl.BlockSpec((_BLK1, DP), lambda i: (i, 0)),
            pl.BlockSpec((_BLK1, 8), lambda i: (i, 0)),
            pl.BlockSpec((1, NH * DP), lambda i: (0, 0)),
        ],
        out_shape=[
            jax.ShapeDtypeStruct((NN, DP), jnp.float32),
            jax.ShapeDtypeStruct((NN, DP), jnp.float32),
            jax.ShapeDtypeStruct((NN, 8), jnp.float32),
            jax.ShapeDtypeStruct((1, NH * DP), jnp.float32),
        ],
    )(x, wcat, bcat, ab)


_CH = EE // 16


def _k2a_body(sn_hbm, vi_hbm, vj_hbm, y0_hbm, y1_hbm, sn_v, vi_v, vj_v, y_v):
    c = lax.axis_index("c")
    s = lax.axis_index("s")
    base = s * _CH
    pltpu.sync_copy(sn_hbm, sn_v)
    pltpu.sync_copy(vi_hbm.at[pl.ds(base, _CH)], vi_v)
    pltpu.sync_copy(vj_hbm.at[pl.ds(base, _CH)], vj_v)
    zi = jnp.zeros((16,), jnp.int32)

    def body(k, _):
        sl = pl.ds(k * 16, 16)
        gi = plsc.load_gather(sn_v, [vi_v[sl] * 4 + c])
        gj = plsc.load_gather(sn_v, [vj_v[sl] * 4 + (NH + c)])
        t = gi + gj
        y_v[sl] = jnp.where(t >= 0.0, t, t * 0.2)
        return ()

    lax.fori_loop(0, _CH // 16, body, ())

    @pl.when(c == 0)
    def _():
        pltpu.sync_copy(y_v, y0_hbm.at[pl.ds(base, _CH)])

    @pl.when(c == 1)
    def _():
        pltpu.sync_copy(y_v, y1_hbm.at[pl.ds(base, _CH)])


def _k2a(sn, vi_s, vj_s):
    mesh = plsc.VectorSubcoreMesh(core_axis_name="c", subcore_axis_name="s")
    f = pl.kernel(
        _k2a_body,
        compiler_params=pltpu.CompilerParams(needs_layout_passes=False),
        out_type=[
            jax.ShapeDtypeStruct((EE,), jnp.float32),
            jax.ShapeDtypeStruct((EE,), jnp.float32),
        ],
        mesh=mesh,
        scratch_types=[
            pltpu.VMEM((NN * 2 * NH,), jnp.float32),
            pltpu.VMEM((_CH,), jnp.int32),
            pltpu.VMEM((_CH,), jnp.int32),
            pltpu.VMEM((_CH,), jnp.float32),
        ],
    )
    return f(sn, vi_s, vj_s)


_BB = 80
_MB = 8
_MC = 16
_RPT = _MB * _MC
_EPW = _RPT * _BB
_ZPAD = 10240


def _k2b_body(wx_hbm, vi2_hbm, vj2_hbm, w0_hbm, w1_hbm, num_hbm, z0_hbm,
              z1_hbm, num_sh, z_sh, zbz_v, vi2_v, vj2_v, vja_v, wv_v,
              rows_a, rows_b, sem_a, sem_b):
    c = lax.axis_index("c")
    s = lax.axis_index("s")
    rows = (rows_a, rows_b)
    sems = (sem_a, sem_b)

    def za_loop(k, _):
        rows_a[k // 8, pl.ds((k % 8) * 16, 16)] = jnp.zeros((16,), jnp.float32)
        return ()

    lax.fori_loop(0, _BB * 8, za_loop, ())

    def zbz_loop(k, _):
        zbz_v[pl.ds(k * 16, 16)] = jnp.zeros((16,), jnp.float32)
        return ()

    lax.fori_loop(0, 40, zbz_loop, ())

    for q in range(7):
        pltpu.sync_copy(rows_a, num_sh.at[pl.ds(s * 625 + q * 80, 80)])
    pltpu.sync_copy(rows_a.at[pl.ds(0, 65)],
                    num_sh.at[pl.ds(s * 625 + 560, 65)])
    pltpu.sync_copy(zbz_v, z_sh.at[pl.ds(s * 640, 640)])
    plsc.subcore_barrier()

    def scale(rref, wofs):
        def body(rr, _):
            r = rr * 4
            zi = jnp.zeros((16,), jnp.int32)
            wbs = [plsc.load_gather(wv_v, [zi + wofs + r + u]) for u in range(4)]
            for p in range(DP // 16):
                sl = pl.ds(p * 16, 16)
                for u in range(4):
                    rref[r + u, sl] = rref[r + u, sl] * wbs[u]
            return ()

        lax.fori_loop(0, _BB // 4, body, ())

    def mc_loop(mc, _):
        rowbase = s * _RPT + mc * _MB
        base = rowbase * _BB
        pltpu.sync_copy(vi2_hbm.at[pl.ds(rowbase, _MB)], vi2_v)
        pltpu.sync_copy(vj2_hbm.at[pl.ds(rowbase, _MB)], vj2_v)

        @pl.when(c == 0)
        def _():
            pltpu.sync_copy(w0_hbm.at[pl.ds(base, _MB * _BB)], wv_v)

        @pl.when(c == 1)
        def _():
            pltpu.sync_copy(w1_hbm.at[pl.ds(base, _MB * _BB)], wv_v)

        def prep(k, _):
            sl = pl.ds((k % 5) * 16, 16)
            vja_v[k // 5, sl] = vj2_v[k // 5, sl] + c * NN
            return ()

        lax.fori_loop(0, _MB * 5, prep, ())

        pltpu.async_copy(wx_hbm.at[vja_v.at[0]], rows_a, sem_a)
        for b in range(_MB):
            pb = b % 2
            if b + 1 < _MB:
                pltpu.async_copy(wx_hbm.at[vja_v.at[b + 1]],
                                 rows[(b + 1) % 2], sems[(b + 1) % 2])
            pltpu.make_async_copy(wx_hbm.at[vja_v.at[b]],
                                  rows[pb], sems[pb]).wait()
            scale(rows[pb], b * _BB)
            pltpu.sync_copy(rows[pb], num_sh.at[vi2_v.at[b]], add=True)
            pltpu.sync_copy(wv_v.at[pl.ds(b * _BB, _BB)],
                            z_sh.at[vi2_v.at[b]], add=True)
        return ()

    lax.fori_loop(0, _MC, mc_loop, ())
    plsc.subcore_barrier()

    @pl.when((s == 0) & (c == 0))
    def _():
        pltpu.sync_copy(num_sh, num_hbm.at[0])
        pltpu.sync_copy(z_sh, z0_hbm)

    @pl.when((s == 0) & (c == 1))
    def _():
        pltpu.sync_copy(num_sh, num_hbm.at[1])
        pltpu.sync_copy(z_sh, z1_hbm)


def _k2b(wxcat, vi2, vj2, w0, w1):
    mesh = plsc.VectorSubcoreMesh(core_axis_name="c", subcore_axis_name="s")
    f = pl.kernel(
        _k2b_body,
        compiler_params=pltpu.CompilerParams(needs_layout_passes=False),
        out_type=[
            jax.ShapeDtypeStruct((NH, NN, DP), jnp.float32),
            jax.ShapeDtypeStruct((_ZPAD,), jnp.float32),
            jax.ShapeDtypeStruct((_ZPAD,), jnp.float32),
        ],
        mesh=mesh,
        scratch_types=[
            pltpu.VMEM_SHARED((NN, DP), jnp.float32),
            pltpu.VMEM_SHARED((_ZPAD,), jnp.float32),
            pltpu.VMEM((640,), jnp.float32),
            pltpu.VMEM((_MB, _BB), jnp.int32),
            pltpu.VMEM((_MB, _BB), jnp.int32),
            pltpu.VMEM((_MB, _BB), jnp.int32),
            pltpu.VMEM((_MB * _BB,), jnp.float32),
            pltpu.VMEM((_BB, DP), jnp.float32),
            pltpu.VMEM((_BB, DP), jnp.float32),
            pltpu.SemaphoreType.DMA,
            pltpu.SemaphoreType.DMA,
        ],
    )
    return f(wxcat, vi2, vj2, w0, w1)


def _k2w_body(v_ref, m_ref, o_ref):
    o_ref[...] = m_ref[...] * (jnp.exp(v_ref[...]) - 1.0)


def _k2w(v2, m):
    vr = v2.reshape(20, NH * EE // 20)
    mr = jnp.broadcast_to(m, (NH, EE)).reshape(20, NH * EE // 20)
    out = pl.pallas_call(
        _k2w_body,
        out_shape=jax.ShapeDtypeStruct(vr.shape, jnp.float32),
    )(vr, mr)
    return out.reshape(NH, EE)


_BLK3 = 1000


def _k3_body(num_ref, z_ref, cs_ref, o_ref):
    v = (cs_ref[0] + num_ref[0]) / (float(NN) + z_ref[0])
    o_ref[...] = jnp.where(v > 0.0, v, jnp.exp(v) - 1.0)


def _k3(num, z3, s2):
    return pl.pallas_call(
        _k3_body,
        grid=(NH, NN // _BLK3),
        in_specs=[
            pl.BlockSpec((1, _BLK3, DP), lambda h, i: (h, i, 0)),
            pl.BlockSpec((1, _BLK3, 1), lambda h, i: (h, i, 0)),
            pl.BlockSpec((1, 1, DP), lambda h, i: (h, 0, 0)),
        ],
        out_specs=pl.BlockSpec((_BLK3, DP), lambda h, i: (i, h)),
        out_shape=jax.ShapeDtypeStruct((NN, NH * DP), jnp.float32),
    )(num, z3, s2)


def kernel(x, edge_index, W_w, W_b, a_w, a_b):
    vi = edge_index[0]
    vj = edge_index[1]

    wcat = jnp.concatenate([W_w[0].T, W_w[1].T], axis=1)
    bcat = W_b.reshape(1, NH * DP)
    zcol = jnp.zeros((DP,), jnp.float32)
    ab = jnp.stack([
        jnp.concatenate([a_w[0, :DP], zcol]),
        jnp.concatenate([zcol, a_w[1, :DP]]),
        jnp.concatenate([a_w[0, DP:], zcol]),
        jnp.concatenate([zcol, a_w[1, DP:]]),
    ], axis=1)
    ab = jnp.concatenate([ab, jnp.zeros((NH * DP, 4), jnp.float32)], axis=1)

    wx0, wx1, s8, cs = _k1(x, wcat, bcat, ab)

    key = vi * NN + vj
    ks = jnp.sort(key)
    vi_s = ks // NN
    vj_s = ks - vi_s * NN

    sn = jnp.stack([
        s8[:, 0], s8[:, 1],
        s8[:, 2] + a_b[0], s8[:, 3] + a_b[1],
    ], axis=1)

    y0, y1 = _k2a(sn.reshape(-1), vi_s, vj_s)
    y2 = jnp.stack([y0, y1])

    starts = jnp.concatenate(
        [jnp.ones((1,), bool), ks[1:] != ks[:-1]])
    lasts = jnp.concatenate([starts[1:], jnp.ones((1,), bool)])
    idxe = jnp.arange(EE, dtype=jnp.int32)
    c0 = jnp.cumsum(y2, axis=1)
    sidx = jax.lax.cummax(jnp.where(starts, idxe, 0))
    cprev = jnp.where(sidx > 0, c0[:, sidx - 1], 0.0)
    m = lasts.astype(jnp.float32)
    w2 = _k2w(c0 - cprev, m)

    wxcat = jnp.concatenate([wx0, wx1], axis=0)
    padi = jnp.zeros((16, 3, _BB), jnp.int32)
    padw = jnp.zeros((NH, 16, 3, _BB), jnp.float32)
    vi2 = jnp.concatenate(
        [vi_s.reshape(16, 125, _BB), padi], axis=1).reshape(16 * 128, _BB)
    vj2 = jnp.concatenate(
        [vj_s.reshape(16, 125, _BB), padi], axis=1).reshape(16 * 128, _BB)
    w2p = jnp.concatenate(
        [w2.reshape(NH, 16, 125, _BB), padw], axis=2).reshape(NH, 16 * 128 * _BB)
    num, z0, z1 = _k2b(wxcat, vi2, vj2, w2p[0], w2p[1])

    z3 = jnp.stack([z0[:NN], z1[:NN]]).reshape(NH, NN, 1)
    s2 = cs.reshape(NH, 1, DP)
    return _k3(num, z3, s2)


# speedup vs baseline: 4.7572x; 1.2036x over previous
import functools

import jax
import jax.numpy as jnp
from jax import lax
from jax.experimental import pallas as pl
from jax.experimental.pallas import tpu as pltpu
from jax.experimental.pallas import tpu_sc as plsc

NN = 10000
EE = 160000
DIN = 256
DP = 128
NH = 2

_BLK1 = 1000


def _k1_body(x_ref, w_ref, b_ref, ab_ref, wx0_ref, wx1_ref, s_ref, cs_ref):
    i = pl.program_id(0)
    xb = x_ref[...]
    wx = jnp.dot(xb, w_ref[...], preferred_element_type=jnp.float32) + b_ref[...]
    wx0_ref[...] = wx[:, :DP]
    wx1_ref[...] = wx[:, DP:]
    s_ref[...] = jnp.dot(wx, ab_ref[...], preferred_element_type=jnp.float32)
    colsum = jnp.sum(wx, axis=0, keepdims=True)

    @pl.when(i == 0)
    def _():
        cs_ref[...] = colsum

    @pl.when(i > 0)
    def _():
        cs_ref[...] += colsum


def _k1(x, wcat, bcat, ab):
    grid = NN // _BLK1
    return pl.pallas_call(
        _k1_body,
        grid=(grid,),
        in_specs=[
            pl.BlockSpec((_BLK1, DIN), lambda i: (i, 0)),
            pl.BlockSpec((DIN, NH * DP), lambda i: (0, 0)),
            pl.BlockSpec((1, NH * DP), lambda i: (0, 0)),
            pl.BlockSpec((DIN, 8), lambda i: (0, 0)),
        ],
        out_specs=[
            pl.BlockSpec((_BLK1, DP), lambda i: (i, 0)),
            pl.BlockSpec((_BLK1, DP), lambda i: (i, 0)),
            pl.BlockSpec((_BLK1, 8), lambda i: (i, 0)),
            pl.BlockSpec((1, NH * DP), lambda i: (0, 0)),
        ],
        out_shape=[
            jax.ShapeDtypeStruct((NN, DP), jnp.float32),
            jax.ShapeDtypeStruct((NN, DP), jnp.float32),
            jax.ShapeDtypeStruct((NN, 8), jnp.float32),
            jax.ShapeDtypeStruct((1, NH * DP), jnp.float32),
        ],
    )(x, wcat, bcat, ab)


_CH = EE // 16


def _k2a_body(sn_hbm, vi_hbm, vj_hbm, y0_hbm, y1_hbm, sn_v, vi_v, vj_v, y_v):
    c = lax.axis_index("c")
    s = lax.axis_index("s")
    base = s * _CH
    pltpu.sync_copy(sn_hbm, sn_v)
    pltpu.sync_copy(vi_hbm.at[pl.ds(base, _CH)], vi_v)
    pltpu.sync_copy(vj_hbm.at[pl.ds(base, _CH)], vj_v)
    zi = jnp.zeros((16,), jnp.int32)

    def body(k, _):
        sl = pl.ds(k * 16, 16)
        gi = plsc.load_gather(sn_v, [vi_v[sl] * 4 + c])
        gj = plsc.load_gather(sn_v, [vj_v[sl] * 4 + (NH + c)])
        t = gi + gj
        y_v[sl] = jnp.where(t >= 0.0, t, t * 0.2)
        return ()

    lax.fori_loop(0, _CH // 16, body, ())

    @pl.when(c == 0)
    def _():
        pltpu.sync_copy(y_v, y0_hbm.at[pl.ds(base, _CH)])

    @pl.when(c == 1)
    def _():
        pltpu.sync_copy(y_v, y1_hbm.at[pl.ds(base, _CH)])


def _k2a(sn, vi_s, vj_s):
    mesh = plsc.VectorSubcoreMesh(core_axis_name="c", subcore_axis_name="s")
    f = pl.kernel(
        _k2a_body,
        compiler_params=pltpu.CompilerParams(needs_layout_passes=False),
        out_type=[
            jax.ShapeDtypeStruct((EE,), jnp.float32),
            jax.ShapeDtypeStruct((EE,), jnp.float32),
        ],
        mesh=mesh,
        scratch_types=[
            pltpu.VMEM((NN * 2 * NH,), jnp.float32),
            pltpu.VMEM((_CH,), jnp.int32),
            pltpu.VMEM((_CH,), jnp.int32),
            pltpu.VMEM((_CH,), jnp.float32),
        ],
    )
    return f(sn, vi_s, vj_s)


_BB = 80
_MB = 8
_MC = 16
_RPT = _MB * _MC
_EPW = _RPT * _BB
_ZPAD = 10240


def _k2b_body(wx_hbm, vi2_hbm, vj2_hbm, w0_hbm, w1_hbm, num_hbm, z0_hbm,
              z1_hbm, num_sh, z_sh, zbz_v, vi2_v, vj2_v, vja_v, wv_v,
              rows_a, rows_b, sem_a, sem_b):
    c = lax.axis_index("c")
    s = lax.axis_index("s")
    rows = (rows_a, rows_b)
    sems = (sem_a, sem_b)

    def za_loop(k, _):
        rows_a[k // 8, pl.ds((k % 8) * 16, 16)] = jnp.zeros((16,), jnp.float32)
        return ()

    lax.fori_loop(0, _BB * 8, za_loop, ())

    def zbz_loop(k, _):
        zbz_v[pl.ds(k * 16, 16)] = jnp.zeros((16,), jnp.float32)
        return ()

    lax.fori_loop(0, 40, zbz_loop, ())

    for q in range(7):
        pltpu.sync_copy(rows_a, num_sh.at[pl.ds(s * 625 + q * 80, 80)])
    pltpu.sync_copy(rows_a.at[pl.ds(0, 65)],
                    num_sh.at[pl.ds(s * 625 + 560, 65)])
    pltpu.sync_copy(zbz_v, z_sh.at[pl.ds(s * 640, 640)])
    plsc.subcore_barrier()

    def scale(rref, wofs):
        def body(rr, _):
            r = rr * 4
            zi = jnp.zeros((16,), jnp.int32)
            wbs = [plsc.load_gather(wv_v, [zi + wofs + r + u]) for u in range(4)]
            for p in range(DP // 16):
                sl = pl.ds(p * 16, 16)
                for u in range(4):
                    rref[r + u, sl] = rref[r + u, sl] * wbs[u]
            return ()

        lax.fori_loop(0, _BB // 4, body, ())

    def mc_loop(mc, _):
        rowbase = s * _RPT + mc * _MB
        base = rowbase * _BB
        pltpu.sync_copy(vi2_hbm.at[pl.ds(rowbase, _MB)], vi2_v)
        pltpu.sync_copy(vj2_hbm.at[pl.ds(rowbase, _MB)], vj2_v)

        @pl.when(c == 0)
        def _():
            pltpu.sync_copy(w0_hbm.at[pl.ds(base, _MB * _BB)], wv_v)

        @pl.when(c == 1)
        def _():
            pltpu.sync_copy(w1_hbm.at[pl.ds(base, _MB * _BB)], wv_v)

        def prep(k, _):
            sl = pl.ds((k % 5) * 16, 16)
            vja_v[k // 5, sl] = vj2_v[k // 5, sl] + c * NN
            return ()

        lax.fori_loop(0, _MB * 5, prep, ())

        pltpu.async_copy(wx_hbm.at[vja_v.at[0]], rows_a, sem_a)
        for b in range(_MB):
            pb = b % 2
            if b + 1 < _MB:
                pltpu.async_copy(wx_hbm.at[vja_v.at[b + 1]],
                                 rows[(b + 1) % 2], sems[(b + 1) % 2])
            pltpu.make_async_copy(wx_hbm.at[vja_v.at[b]],
                                  rows[pb], sems[pb]).wait()
            scale(rows[pb], b * _BB)
            pltpu.sync_copy(rows[pb], num_sh.at[vi2_v.at[b]], add=True)
            pltpu.sync_copy(wv_v.at[pl.ds(b * _BB, _BB)],
                            z_sh.at[vi2_v.at[b]], add=True)
        return ()

    lax.fori_loop(0, _MC, mc_loop, ())
    plsc.subcore_barrier()

    @pl.when((s == 0) & (c == 0))
    def _():
        pltpu.sync_copy(num_sh, num_hbm.at[0])
        pltpu.sync_copy(z_sh, z0_hbm)

    @pl.when((s == 0) & (c == 1))
    def _():
        pltpu.sync_copy(num_sh, num_hbm.at[1])
        pltpu.sync_copy(z_sh, z1_hbm)


def _k2b(wxcat, vi2, vj2, w0, w1):
    mesh = plsc.VectorSubcoreMesh(core_axis_name="c", subcore_axis_name="s")
    f = pl.kernel(
        _k2b_body,
        compiler_params=pltpu.CompilerParams(needs_layout_passes=False),
        out_type=[
            jax.ShapeDtypeStruct((NH, NN, DP), jnp.float32),
            jax.ShapeDtypeStruct((_ZPAD,), jnp.float32),
            jax.ShapeDtypeStruct((_ZPAD,), jnp.float32),
        ],
        mesh=mesh,
        scratch_types=[
            pltpu.VMEM_SHARED((NN, DP), jnp.float32),
            pltpu.VMEM_SHARED((_ZPAD,), jnp.float32),
            pltpu.VMEM((640,), jnp.float32),
            pltpu.VMEM((_MB, _BB), jnp.int32),
            pltpu.VMEM((_MB, _BB), jnp.int32),
            pltpu.VMEM((_MB, _BB), jnp.int32),
            pltpu.VMEM((_MB * _BB,), jnp.float32),
            pltpu.VMEM((_BB, DP), jnp.float32),
            pltpu.VMEM((_BB, DP), jnp.float32),
            pltpu.SemaphoreType.DMA,
            pltpu.SemaphoreType.DMA,
        ],
    )
    return f(wxcat, vi2, vj2, w0, w1)


def _k2w_body(v_ref, m_ref, o_ref):
    o_ref[...] = m_ref[...] * (jnp.exp(v_ref[...]) - 1.0)


def _k2w(v2, m):
    vr = v2.reshape(20, NH * EE // 20)
    mr = jnp.broadcast_to(m, (NH, EE)).reshape(20, NH * EE // 20)
    out = pl.pallas_call(
        _k2w_body,
        out_shape=jax.ShapeDtypeStruct(vr.shape, jnp.float32),
    )(vr, mr)
    return out.reshape(NH, EE)


_BLK3 = 1000


def _k3_body(num_ref, z_ref, cs_ref, o_ref):
    v = (cs_ref[0] + num_ref[0]) / (float(NN) + z_ref[0])
    o_ref[...] = jnp.where(v > 0.0, v, jnp.exp(v) - 1.0)


def _k3(num, z3, s2):
    return pl.pallas_call(
        _k3_body,
        grid=(NH, NN // _BLK3),
        in_specs=[
            pl.BlockSpec((1, _BLK3, DP), lambda h, i: (h, i, 0)),
            pl.BlockSpec((1, _BLK3, 1), lambda h, i: (h, i, 0)),
            pl.BlockSpec((1, 1, DP), lambda h, i: (h, 0, 0)),
        ],
        out_specs=pl.BlockSpec((_BLK3, DP), lambda h, i: (i, h)),
        out_shape=jax.ShapeDtypeStruct((NN, NH * DP), jnp.float32),
    )(num, z3, s2)


def kernel(x, edge_index, W_w, W_b, a_w, a_b):
    vi = edge_index[0]
    vj = edge_index[1]

    wcat = jnp.concatenate([W_w[0].T, W_w[1].T], axis=1)
    bcat = W_b.reshape(1, NH * DP)
    zcol = jnp.zeros((DP,), jnp.float32)
    ab = jnp.stack([
        jnp.concatenate([a_w[0, :DP], zcol]),
        jnp.concatenate([zcol, a_w[1, :DP]]),
        jnp.concatenate([a_w[0, DP:], zcol]),
        jnp.concatenate([zcol, a_w[1, DP:]]),
    ], axis=1)
    ab = jnp.concatenate([ab, jnp.zeros((NH * DP, 4), jnp.float32)], axis=1)

    wx0, wx1, s8, cs = _k1(x, wcat, bcat, ab)

    key = vi * NN + vj
    ks = lax.sort([key], is_stable=False)[0]
    vi_s = ks // NN
    vj_s = ks - vi_s * NN

    sn = jnp.stack([
        s8[:, 0], s8[:, 1],
        s8[:, 2] + a_b[0], s8[:, 3] + a_b[1],
    ], axis=1)

    y0, y1 = _k2a(sn.reshape(-1), vi_s, vj_s)
    y2 = jnp.stack([y0, y1])

    starts = jnp.concatenate(
        [jnp.ones((1,), bool), ks[1:] != ks[:-1]])
    lasts = jnp.concatenate([starts[1:], jnp.ones((1,), bool)])
    idxe = jnp.arange(EE, dtype=jnp.int32)
    c0 = jnp.cumsum(y2, axis=1)
    sidx = jax.lax.cummax(jnp.where(starts, idxe, 0))
    cprev = jnp.where(sidx > 0, c0[:, sidx - 1], 0.0)
    m = lasts.astype(jnp.float32)
    w2 = _k2w(c0 - cprev, m)

    wxcat = jnp.concatenate([wx0, wx1], axis=0)
    padi = jnp.zeros((16, 3, _BB), jnp.int32)
    padw = jnp.zeros((NH, 16, 3, _BB), jnp.float32)
    vi2 = jnp.concatenate(
        [vi_s.reshape(16, 125, _BB), padi], axis=1).reshape(16 * 128, _BB)
    vj2 = jnp.concatenate(
        [vj_s.reshape(16, 125, _BB), padi], axis=1).reshape(16 * 128, _BB)
    w2p = jnp.concatenate(
        [w2.reshape(NH, 16, 125, _BB), padw], axis=2).reshape(NH, 16 * 128 * _BB)
    num, z0, z1 = _k2b(wxcat, vi2, vj2, w2p[0], w2p[1])

    z3 = jnp.stack([z0[:NN], z1[:NN]]).reshape(NH, NN, 1)
    s2 = cs.reshape(NH, 1, DP)
    return _k3(num, z3, s2)
